# Initial kernel scaffold; baseline (speedup 1.0000x reference)
import jax
import jax.numpy as jnp
from jax.experimental import pallas as pl


def kernel(triples, entity_table, relation_table):
    raise NotImplementedError("write your pallas kernel here")


# speedup vs baseline: 1.2575x; 1.2575x over previous
import functools

import jax
import jax.numpy as jnp
from jax import lax
from jax.experimental import pallas as pl
from jax.experimental.pallas import tpu as pltpu
from jax.experimental.pallas import tpu_sc as plsc

B = 16384
D = 128
NC = 2
NS = 16
NW = NC * NS
BPW = B // NW
CHUNK = 256
NCHUNK = BPW // CHUNK

_mesh = plsc.VectorSubcoreMesh(core_axis_name="c", subcore_axis_name="s")


@functools.partial(
    pl.kernel,
    mesh=_mesh,
    out_type=jax.ShapeDtypeStruct((B,), jnp.float32),
    scratch_types=[
        pltpu.VMEM((CHUNK,), jnp.int32),
        pltpu.VMEM((CHUNK,), jnp.int32),
        pltpu.VMEM((CHUNK,), jnp.int32),
        pltpu.VMEM((CHUNK, D), jnp.float32),
        pltpu.VMEM((CHUNK, D), jnp.float32),
        pltpu.VMEM((CHUNK, D), jnp.float32),
        pltpu.VMEM((CHUNK,), jnp.float32),
        pltpu.SemaphoreType.DMA,
    ],
)
def _score_kernel(hidx_hbm, ridx_hbm, tidx_hbm, etab_hbm, rtab_hbm, out_hbm,
                  hi_v, ri_v, ti_v, h_v, r_v, t_v, s_v, sem):
    wid = lax.axis_index("s") * NC + lax.axis_index("c")
    base = wid * BPW

    def chunk_body(c, carry):
        off = base + c * CHUNK
        pltpu.sync_copy(hidx_hbm.at[pl.ds(off, CHUNK)], hi_v)
        pltpu.sync_copy(ridx_hbm.at[pl.ds(off, CHUNK)], ri_v)
        pltpu.sync_copy(tidx_hbm.at[pl.ds(off, CHUNK)], ti_v)
        ch = pltpu.async_copy(etab_hbm.at[hi_v], h_v, sem)
        cr = pltpu.async_copy(rtab_hbm.at[ri_v], r_v, sem)
        ct = pltpu.async_copy(etab_hbm.at[ti_v], t_v, sem)
        ch.wait()
        cr.wait()
        ct.wait()

        lane = lax.iota(jnp.int32, 16)
        perms = [lane ^ k for k in (1, 2, 4, 8)]
        onehot = [lane == l for l in range(16)]

        def group_body(g, gc):
            ro = g * 16
            scores = jnp.zeros((16,), jnp.float32)
            for l in range(16):
                i = ro + l
                a = h_v[i, pl.ds(0, 16)] * r_v[i, pl.ds(0, 16)] * t_v[i, pl.ds(0, 16)]
                for j in range(1, 8):
                    sl = pl.ds(j * 16, 16)
                    a = a + h_v[i, sl] * r_v[i, sl] * t_v[i, sl]
                for p in perms:
                    a = a + a.at[p].get(mode="promise_in_bounds")
                scores = jnp.where(onehot[l], a, scores)
            s_v[pl.ds(ro, 16)] = scores
            return gc

        lax.fori_loop(0, CHUNK // 16, group_body, 0)
        pltpu.sync_copy(s_v, out_hbm.at[pl.ds(off, CHUNK)])
        return carry

    lax.fori_loop(0, NCHUNK, chunk_body, 0)


def kernel(triples, entity_table, relation_table):
    t32 = triples.astype(jnp.int32)
    return _score_kernel(t32[:, 0], t32[:, 1], t32[:, 2],
                         entity_table, relation_table)


# speedup vs baseline: 2.4713x; 1.9653x over previous
import functools

import jax
import jax.numpy as jnp
from jax import lax
from jax.experimental import pallas as pl
from jax.experimental.pallas import tpu as pltpu
from jax.experimental.pallas import tpu_sc as plsc

B = 16384
D = 128
NC = 2
NS = 16
NW = NC * NS
BPW = B // NW
CHUNK = 64
NCHUNK = BPW // CHUNK

_mesh = plsc.VectorSubcoreMesh(core_axis_name="c", subcore_axis_name="s")


@functools.partial(
    pl.kernel,
    mesh=_mesh,
    out_type=jax.ShapeDtypeStruct((B,), jnp.float32),
    scratch_types=[
        pltpu.VMEM((BPW,), jnp.int32),
        pltpu.VMEM((BPW,), jnp.int32),
        pltpu.VMEM((BPW,), jnp.int32),
        pltpu.VMEM((CHUNK, D), jnp.float32),
        pltpu.VMEM((CHUNK, D), jnp.float32),
        pltpu.VMEM((CHUNK, D), jnp.float32),
        pltpu.VMEM((CHUNK, D), jnp.float32),
        pltpu.VMEM((CHUNK, D), jnp.float32),
        pltpu.VMEM((CHUNK, D), jnp.float32),
        pltpu.VMEM((BPW,), jnp.float32),
        pltpu.SemaphoreType.DMA,
        pltpu.SemaphoreType.DMA,
    ],
)
def _score_kernel(hidx_hbm, ridx_hbm, tidx_hbm, etab_hbm, rtab_hbm, out_hbm,
                  hi_v, ri_v, ti_v, hA, rA, tA, hB, rB, tB, s_v, semA, semB):
    wid = lax.axis_index("s") * NC + lax.axis_index("c")
    base = wid * BPW

    pltpu.sync_copy(hidx_hbm.at[pl.ds(base, BPW)], hi_v)
    pltpu.sync_copy(ridx_hbm.at[pl.ds(base, BPW)], ri_v)
    pltpu.sync_copy(tidx_hbm.at[pl.ds(base, BPW)], ti_v)

    bufs = [(hA, rA, tA), (hB, rB, tB)]
    sems = [semA, semB]
    pend = [None, None]

    def issue(c, slot):
        sl = pl.ds(c * CHUNK, CHUNK)
        hb, rb, tb = bufs[slot]
        sem = sems[slot]
        ch = pltpu.async_copy(etab_hbm.at[hi_v.at[sl]], hb, sem)
        cr = pltpu.async_copy(rtab_hbm.at[ri_v.at[sl]], rb, sem)
        ct = pltpu.async_copy(etab_hbm.at[ti_v.at[sl]], tb, sem)
        return (ch, cr, ct)

    lane = lax.iota(jnp.int32, 16)
    perms = [lane ^ k for k in (1, 2, 4, 8)]

    pend[0] = issue(0, 0)

    for c in range(NCHUNK):
        cur = c % 2
        nxt = (c + 1) % 2
        if c + 1 < NCHUNK:
            pend[nxt] = issue(c + 1, nxt)
        for cp in pend[cur]:
            cp.wait()
        h_v, r_v, t_v = bufs[cur]
        coff = c * CHUNK

        def group_body(g, gc):
            def row_body(l, scores):
                i = g * 16 + l
                a = h_v[i, pl.ds(0, 16)] * r_v[i, pl.ds(0, 16)] * t_v[i, pl.ds(0, 16)]
                for j in range(1, 8):
                    sl = pl.ds(j * 16, 16)
                    a = a + h_v[i, sl] * r_v[i, sl] * t_v[i, sl]
                for p in perms:
                    a = a + a.at[p].get(mode="promise_in_bounds")
                return jnp.where(lane == l, a, scores)

            scores = lax.fori_loop(0, 16, row_body, jnp.zeros((16,), jnp.float32))
            s_v[pl.ds(coff + g * 16, 16)] = scores
            return gc

        lax.fori_loop(0, CHUNK // 16, group_body, 0)

    pltpu.sync_copy(s_v, out_hbm.at[pl.ds(base, BPW)])


def kernel(triples, entity_table, relation_table):
    t32 = triples.astype(jnp.int32)
    return _score_kernel(t32[:, 0], t32[:, 1], t32[:, 2],
                         entity_table, relation_table)
